# baseline (speedup 1.0000x reference)
import functools

import numpy as np
import jax
import jax.numpy as jnp
from jax import lax
from jax.experimental import pallas as pl
from jax.experimental.pallas import tpu as pltpu
from jax.experimental.pallas import tpu_sc as plsc

VOCAB = 100000
DIM = 64
B = 4096
L = 200

_NCHUNK = DIM // 16


def _sc_sums(pan_idx, inst_idx, sem_idx, pan_tab, inst_tab, sem_tab):
    info = plsc.get_sparse_core_info()
    nc, ns = info.num_cores, info.num_subcores
    nw = nc * ns
    rows_per_w = B // nw

    mesh = plsc.VectorSubcoreMesh(core_axis_name="c", subcore_axis_name="s")

    @functools.partial(
        pl.kernel,
        mesh=mesh,
        compiler_params=pltpu.CompilerParams(
            use_tc_tiling_on_sc=False, needs_layout_passes=False),
        out_type=jax.ShapeDtypeStruct((3, B, DIM), jnp.float32),
        scratch_types=[
            pltpu.VMEM((rows_per_w * L,), jnp.int32),
            pltpu.VMEM((L, DIM), jnp.bfloat16),
            pltpu.VMEM((L, DIM), jnp.bfloat16),
            pltpu.VMEM((L, DIM), jnp.bfloat16),
            pltpu.VMEM((L, DIM), jnp.bfloat16),
            pltpu.VMEM((rows_per_w, DIM), jnp.float32),
            pltpu.SemaphoreType.DMA,
            pltpu.SemaphoreType.DMA,
            pltpu.SemaphoreType.DMA,
            pltpu.SemaphoreType.DMA,
        ],
    )
    def sums_kernel(idx_hbm, tab_hbm, out_hbm,
                    idx_all, buf0, buf1, buf2, buf3, sums_v,
                    sem0, sem1, sem2, sem3):
        wid = lax.axis_index("s") * nc + lax.axis_index("c")
        base = wid * rows_per_w
        bufs = (buf0, buf1, buf2, buf3)
        sems = (sem0, sem1, sem2, sem3)
        nbuf = 4

        for t in range(3):

            pltpu.sync_copy(
                idx_hbm.at[pl.ds((t * B + base) * L, rows_per_w * L)],
                idx_all)

            def start_gather(i, slot, tab_hbm=tab_hbm):
                pltpu.async_copy(
                    tab_hbm.at[idx_all.at[pl.ds(i * L, 128)]],
                    bufs[slot].at[pl.ds(0, 128), :], sems[slot])
                pltpu.async_copy(
                    tab_hbm.at[idx_all.at[pl.ds(i * L + 128, L - 128)]],
                    bufs[slot].at[pl.ds(128, L - 128), :], sems[slot])

            for p in range(nbuf - 1):
                start_gather(p, p)

            def pair_body(g, _, tab_hbm=tab_hbm):
                for s in range(nbuf):
                    i = nbuf * g + s

                    @pl.when(i + nbuf - 1 < rows_per_w)
                    def _(i=i, s=s):
                        start_gather(i + nbuf - 1, (s + nbuf - 1) % nbuf)

                    pltpu.make_async_copy(
                        tab_hbm.at[pl.ds(0, L), :], bufs[s], sems[s]).wait()

                    buf = bufs[s]

                    def acc_body(r, accs, buf=buf):
                        new = []
                        for c in range(DIM // 32):
                            x = buf[r, pl.ds(32 * c, 32)]
                            ae, ao = plsc.unpack(
                                x, format=plsc.PackFormat.INTERLEAVED)
                            new.append(accs[2 * c] + ae)
                            new.append(accs[2 * c + 1] + ao)
                        return tuple(new)

                    zero = jnp.zeros((16,), jnp.float32)
                    accs = lax.fori_loop(0, L, acc_body, (zero,) * _NCHUNK,
                                         unroll=8)
                    for j in range(_NCHUNK):
                        sums_v[i, pl.ds(16 * j, 16)] = accs[j]
                return 0

            lax.fori_loop(0, rows_per_w // nbuf, pair_body, 0)
            pltpu.sync_copy(sums_v, out_hbm.at[t, pl.ds(base, rows_per_w), :])

    idx_flat = (jnp.stack([pan_idx, inst_idx + VOCAB, sem_idx + 2 * VOCAB])
                .reshape(3 * B * L))
    big_tab = jnp.concatenate(
        [pan_tab, inst_tab, sem_tab], axis=0).astype(jnp.bfloat16)
    return sums_kernel(idx_flat, big_tab)


def _proj_body(s_ref, wt_ref, b3_ref, o_ref):
    wt = wt_ref[...]
    outs = [
        jnp.dot(s_ref[t], wt, preferred_element_type=jnp.float32)
        for t in range(3)
    ]
    o_ref[...] = jnp.concatenate(outs, axis=-1) + b3_ref[...]


def _proj(sums, wt, b3):
    blk = 512
    return pl.pallas_call(
        _proj_body,
        grid=(B // blk,),
        in_specs=[
            pl.BlockSpec((3, blk, DIM), lambda i: (0, i, 0)),
            pl.BlockSpec((DIM, DIM), lambda i: (0, 0)),
            pl.BlockSpec((1, 3 * DIM), lambda i: (0, 0)),
        ],
        out_specs=pl.BlockSpec((blk, 3 * DIM), lambda i: (i, 0)),
        out_shape=jax.ShapeDtypeStruct((B, 3 * DIM), jnp.float32),
    )(sums, wt, b3)


def kernel(panoptic_text, instance_text, semantic_text, pan_table, inst_table,
           sem_table, W, b):
    pan_idx = panoptic_text.astype(jnp.int32)
    inst_idx = instance_text.astype(jnp.int32)
    sem_idx = semantic_text.astype(jnp.int32)

    sums = _sc_sums(pan_idx, inst_idx, sem_idx, pan_table, inst_table,
                    sem_table)

    perm = np.concatenate([
        np.arange(0, 32, 2), np.arange(1, 32, 2),
        np.arange(32, 64, 2), np.arange(33, 64, 2)])
    wt = (W.T / jnp.float32(L)).astype(jnp.float32)[perm]
    b3 = jnp.tile(b, 3).reshape(1, 3 * DIM).astype(jnp.float32)
    out2d = _proj(sums, wt, b3)
    return out2d.reshape(B, 3, DIM)


# speedup vs baseline: 1.1905x; 1.1905x over previous
import functools

import numpy as np
import jax
import jax.numpy as jnp
from jax import lax
from jax.experimental import pallas as pl
from jax.experimental.pallas import tpu as pltpu
from jax.experimental.pallas import tpu_sc as plsc

VOCAB = 100000
DIM = 64
B = 4096
L = 200

_NCHUNK = DIM // 16


def _sc_sums(pan_idx, inst_idx, sem_idx, pan_tab, inst_tab, sem_tab):
    info = plsc.get_sparse_core_info()
    nc, ns = info.num_cores, info.num_subcores
    nw = nc * ns
    rows_per_w = B // nw

    mesh = plsc.VectorSubcoreMesh(core_axis_name="c", subcore_axis_name="s")

    @functools.partial(
        pl.kernel,
        mesh=mesh,
        compiler_params=pltpu.CompilerParams(
            use_tc_tiling_on_sc=False, needs_layout_passes=False),
        out_type=jax.ShapeDtypeStruct((3, B, DIM), jnp.float32),
        scratch_types=[
            pltpu.VMEM((rows_per_w * L,), jnp.int32),
            pltpu.VMEM((L, DIM), jnp.bfloat16),
            pltpu.VMEM((L, DIM), jnp.bfloat16),
            pltpu.VMEM((L, DIM), jnp.bfloat16),
            pltpu.VMEM((L, DIM), jnp.bfloat16),
            pltpu.VMEM((rows_per_w, DIM), jnp.float32),
            pltpu.SemaphoreType.DMA,
            pltpu.SemaphoreType.DMA,
            pltpu.SemaphoreType.DMA,
            pltpu.SemaphoreType.DMA,
        ],
    )
    def sums_kernel(idx_hbm, pan_t, inst_t, sem_t, out_hbm,
                    idx_all, buf0, buf1, buf2, buf3, sums_v,
                    sem0, sem1, sem2, sem3):
        wid = lax.axis_index("s") * nc + lax.axis_index("c")
        base = wid * rows_per_w
        bufs = (buf0, buf1, buf2, buf3)
        sems = (sem0, sem1, sem2, sem3)
        nbuf = 4

        for t, tab_hbm in enumerate((pan_t, inst_t, sem_t)):

            pltpu.sync_copy(
                idx_hbm.at[pl.ds((t * B + base) * L, rows_per_w * L)],
                idx_all)

            def start_gather(i, slot, tab_hbm=tab_hbm):
                pltpu.async_copy(
                    tab_hbm.at[idx_all.at[pl.ds(i * L, 128)]],
                    bufs[slot].at[pl.ds(0, 128), :], sems[slot])
                pltpu.async_copy(
                    tab_hbm.at[idx_all.at[pl.ds(i * L + 128, L - 128)]],
                    bufs[slot].at[pl.ds(128, L - 128), :], sems[slot])

            for p in range(nbuf - 1):
                start_gather(p, p)

            def pair_body(g, _, tab_hbm=tab_hbm):
                for s in range(nbuf):
                    i = nbuf * g + s

                    @pl.when(i + nbuf - 1 < rows_per_w)
                    def _(i=i, s=s):
                        start_gather(i + nbuf - 1, (s + nbuf - 1) % nbuf)

                    pltpu.make_async_copy(
                        tab_hbm.at[pl.ds(0, L), :], bufs[s], sems[s]).wait()

                    buf = bufs[s]

                    def acc_body(r, accs, buf=buf):
                        new = []
                        for c in range(DIM // 32):
                            x = buf[r, pl.ds(32 * c, 32)]
                            ae, ao = plsc.unpack(
                                x, format=plsc.PackFormat.INTERLEAVED)
                            new.append(accs[2 * c] + ae)
                            new.append(accs[2 * c + 1] + ao)
                        return tuple(new)

                    zero = jnp.zeros((16,), jnp.float32)
                    accs = lax.fori_loop(0, L, acc_body, (zero,) * _NCHUNK,
                                         unroll=8)
                    for j in range(_NCHUNK):
                        sums_v[i, pl.ds(16 * j, 16)] = accs[j]
                return 0

            lax.fori_loop(0, rows_per_w // nbuf, pair_body, 0)
            pltpu.sync_copy(sums_v, out_hbm.at[t, pl.ds(base, rows_per_w), :])

    idx_flat = (jnp.stack([pan_idx, inst_idx, sem_idx]).reshape(3 * B * L))
    return sums_kernel(idx_flat,
                       pan_tab.astype(jnp.bfloat16),
                       inst_tab.astype(jnp.bfloat16),
                       sem_tab.astype(jnp.bfloat16))


def _proj_body(s_ref, wt_ref, b3_ref, o_ref):
    wt = wt_ref[...]
    outs = [
        jnp.dot(s_ref[t], wt, preferred_element_type=jnp.float32)
        for t in range(3)
    ]
    o_ref[...] = jnp.concatenate(outs, axis=-1) + b3_ref[...]


def _proj(sums, wt, b3):
    blk = 512
    return pl.pallas_call(
        _proj_body,
        grid=(B // blk,),
        in_specs=[
            pl.BlockSpec((3, blk, DIM), lambda i: (0, i, 0)),
            pl.BlockSpec((DIM, DIM), lambda i: (0, 0)),
            pl.BlockSpec((1, 3 * DIM), lambda i: (0, 0)),
        ],
        out_specs=pl.BlockSpec((blk, 3 * DIM), lambda i: (i, 0)),
        out_shape=jax.ShapeDtypeStruct((B, 3 * DIM), jnp.float32),
    )(sums, wt, b3)


def kernel(panoptic_text, instance_text, semantic_text, pan_table, inst_table,
           sem_table, W, b):
    pan_idx = panoptic_text.astype(jnp.int32)
    inst_idx = instance_text.astype(jnp.int32)
    sem_idx = semantic_text.astype(jnp.int32)

    sums = _sc_sums(pan_idx, inst_idx, sem_idx, pan_table, inst_table,
                    sem_table)

    perm = np.concatenate([
        np.arange(0, 32, 2), np.arange(1, 32, 2),
        np.arange(32, 64, 2), np.arange(33, 64, 2)])
    wt = (W.T / jnp.float32(L)).astype(jnp.float32)[perm]
    b3 = jnp.tile(b, 3).reshape(1, 3 * DIM).astype(jnp.float32)
    out2d = _proj(sums, wt, b3)
    return out2d.reshape(B, 3, DIM)


# speedup vs baseline: 1.2292x; 1.0325x over previous
import functools

import numpy as np
import jax
import jax.numpy as jnp
from jax import lax
from jax.experimental import pallas as pl
from jax.experimental.pallas import tpu as pltpu
from jax.experimental.pallas import tpu_sc as plsc

VOCAB = 100000
DIM = 64
B = 4096
L = 200

_NCHUNK = DIM // 16


def _make_sums_kernel():
    info = plsc.get_sparse_core_info()
    nc, ns = info.num_cores, info.num_subcores
    nw = nc * ns
    rows_per_w = B // nw

    mesh = plsc.VectorSubcoreMesh(core_axis_name="c", subcore_axis_name="s")

    @functools.partial(
        pl.kernel,
        mesh=mesh,
        compiler_params=pltpu.CompilerParams(
            use_tc_tiling_on_sc=False, needs_layout_passes=False),
        out_type=jax.ShapeDtypeStruct((B, DIM), jnp.float32),
        scratch_types=[
            pltpu.VMEM((rows_per_w * L,), jnp.int32),
            pltpu.VMEM((L, DIM), jnp.bfloat16),
            pltpu.VMEM((L, DIM), jnp.bfloat16),
            pltpu.VMEM((L, DIM), jnp.bfloat16),
            pltpu.VMEM((L, DIM), jnp.bfloat16),
            pltpu.VMEM((rows_per_w, DIM), jnp.float32),
            pltpu.SemaphoreType.DMA,
            pltpu.SemaphoreType.DMA,
            pltpu.SemaphoreType.DMA,
            pltpu.SemaphoreType.DMA,
        ],
    )
    def sums_kernel(idx_hbm, tab_hbm, out_hbm,
                    idx_all, buf0, buf1, buf2, buf3, sums_v,
                    sem0, sem1, sem2, sem3):
        wid = lax.axis_index("s") * nc + lax.axis_index("c")
        base = wid * rows_per_w
        bufs = (buf0, buf1, buf2, buf3)
        sems = (sem0, sem1, sem2, sem3)
        nbuf = 4

        pltpu.sync_copy(
            idx_hbm.at[pl.ds(base * L, rows_per_w * L)], idx_all)

        def start_gather(i, slot):
            pltpu.async_copy(
                tab_hbm.at[idx_all.at[pl.ds(i * L, 128)]],
                bufs[slot].at[pl.ds(0, 128), :], sems[slot])
            pltpu.async_copy(
                tab_hbm.at[idx_all.at[pl.ds(i * L + 128, L - 128)]],
                bufs[slot].at[pl.ds(128, L - 128), :], sems[slot])

        for p in range(nbuf - 1):
            start_gather(p, p)

        def ring_body(g, _):
            for s in range(nbuf):
                i = nbuf * g + s

                @pl.when(i + nbuf - 1 < rows_per_w)
                def _(i=i, s=s):
                    start_gather(i + nbuf - 1, (s + nbuf - 1) % nbuf)

                pltpu.make_async_copy(
                    tab_hbm.at[pl.ds(0, L), :], bufs[s], sems[s]).wait()

                buf = bufs[s]

                def acc_body(r, accs, buf=buf):
                    new = []
                    for c in range(DIM // 32):
                        x = buf[r, pl.ds(32 * c, 32)]
                        ae, ao = plsc.unpack(
                            x, format=plsc.PackFormat.INTERLEAVED)
                        new.append(accs[2 * c] + ae)
                        new.append(accs[2 * c + 1] + ao)
                    return tuple(new)

                zero = jnp.zeros((16,), jnp.float32)
                accs = lax.fori_loop(0, L, acc_body, (zero,) * _NCHUNK,
                                     unroll=8)
                for j in range(_NCHUNK):
                    sums_v[i, pl.ds(16 * j, 16)] = accs[j]
            return 0

        lax.fori_loop(0, rows_per_w // nbuf, ring_body, 0)
        pltpu.sync_copy(sums_v, out_hbm.at[pl.ds(base, rows_per_w)])

    return sums_kernel


def _sc_sums(pan_idx, inst_idx, sem_idx, pan_tab, inst_tab, sem_tab):
    sums_kernel = _make_sums_kernel()
    return [
        sums_kernel(idx.reshape(B * L), tab.astype(jnp.bfloat16))
        for idx, tab in ((pan_idx, pan_tab), (inst_idx, inst_tab),
                         (sem_idx, sem_tab))
    ]


def _proj_body(s0_ref, s1_ref, s2_ref, wt_ref, b3_ref, o_ref):
    wt = wt_ref[...]
    outs = [
        jnp.dot(s_ref[...], wt, preferred_element_type=jnp.float32)
        for s_ref in (s0_ref, s1_ref, s2_ref)
    ]
    o_ref[...] = jnp.concatenate(outs, axis=-1) + b3_ref[...]


def _proj(sums, wt, b3):
    blk = 512
    sblock = pl.BlockSpec((blk, DIM), lambda i: (i, 0))
    return pl.pallas_call(
        _proj_body,
        grid=(B // blk,),
        in_specs=[
            sblock, sblock, sblock,
            pl.BlockSpec((DIM, DIM), lambda i: (0, 0)),
            pl.BlockSpec((1, 3 * DIM), lambda i: (0, 0)),
        ],
        out_specs=pl.BlockSpec((blk, 3 * DIM), lambda i: (i, 0)),
        out_shape=jax.ShapeDtypeStruct((B, 3 * DIM), jnp.float32),
    )(*sums, wt, b3)


def kernel(panoptic_text, instance_text, semantic_text, pan_table, inst_table,
           sem_table, W, b):
    pan_idx = panoptic_text.astype(jnp.int32)
    inst_idx = instance_text.astype(jnp.int32)
    sem_idx = semantic_text.astype(jnp.int32)

    sums = _sc_sums(pan_idx, inst_idx, sem_idx, pan_table, inst_table,
                    sem_table)

    perm = np.concatenate([
        np.arange(0, 32, 2), np.arange(1, 32, 2),
        np.arange(32, 64, 2), np.arange(33, 64, 2)])
    wt = (W.T / jnp.float32(L)).astype(jnp.float32)[perm]
    b3 = jnp.tile(b, 3).reshape(1, 3 * DIM).astype(jnp.float32)
    out2d = _proj(sums, wt, b3)
    return out2d.reshape(B, 3, DIM)
